# baseline (speedup 1.0000x reference)
import functools

import jax
import jax.numpy as jnp
from jax import lax
from jax.experimental import pallas as pl
from jax.experimental.pallas import tpu as pltpu
from jax.experimental.pallas import tpu_sc as plsc

B = 16384
L = 50
D = 64
BT = B * L
V2 = 2000000

_info = plsc.get_sparse_core_info()
NC = _info.num_cores
NS = _info.num_subcores
NW = NC * NS
BPW_B = B // NW

CB = 16
C = CB * L
NCHUNK = BPW_B // CB
LH = L // 2

_mesh = plsc.VectorSubcoreMesh(core_axis_name="c", subcore_axis_name="s")


@functools.partial(
    pl.kernel,
    mesh=_mesh,
    out_type=jax.ShapeDtypeStruct((L, D, B), jnp.float32),
    scratch_types=[
        pltpu.VMEM((C,), jnp.int32),
        pltpu.VMEM((C,), jnp.int32),
        pltpu.VMEM((C, D), jnp.float32),
        pltpu.VMEM((C, D), jnp.float32),
        pltpu.VMEM((LH, D, CB), jnp.float32),
        pltpu.SemaphoreType.DMA,
        pltpu.SemaphoreType.DMA,
        pltpu.SemaphoreType.DMA,
        pltpu.SemaphoreType.DMA,
        pltpu.SemaphoreType.DMA,
    ],
    compiler_params=pltpu.CompilerParams(
        use_tc_tiling_on_sc=False, needs_layout_passes=False),
)
def _embed(idx_hbm, w_hbm, out_hbm,
           idx0, idx1, rows0, rows1, t_v, si0, si1, sg0, sg1, so):
    wid = lax.axis_index("s") * NC + lax.axis_index("c")
    base_b = wid * BPW_B
    idx_v = (idx0, idx1)
    rows_v = (rows0, rows1)
    s_i = (si0, si1)
    s_g = (sg0, sg1)

    lane = lax.iota(jnp.int32, 16)
    d_idx = [lane + 16 * k for k in range(4)]

    pltpu.sync_copy(idx_hbm.at[pl.ds(base_b * L, C)], idx0)
    pltpu.async_copy(w_hbm.at[idx0], rows0, sg0)
    pltpu.async_copy(idx_hbm.at[pl.ds(base_b * L + C, C)], idx1, si1)

    def transpose_half(buf, half, b0):
        def per_l(ll, carry):
            l_idx = jnp.full((16,), ll, jnp.int32)
            l_glob = half * LH + ll
            for b_l in range(CB):
                r = b_l * L + l_glob
                b_idx = jnp.full((16,), b_l, jnp.int32)
                for k in range(4):
                    v = rows_v[buf][r, pl.ds(16 * k, 16)]
                    plsc.store_scatter(t_v, [l_idx, d_idx[k], b_idx], v)
            return carry

        lax.fori_loop(0, LH, per_l, 0)
        cp = pltpu.make_async_copy(
            t_v,
            out_hbm.at[pl.ds(half * LH, LH), :, pl.ds(b0, CB)],
            so)
        cp.start()
        cp.wait()

    def body(h, carry):
        for b in range(2):
            g = 2 * h + b
            b0 = base_b + g * CB
            nb = 1 - b
            pltpu.make_async_copy(w_hbm.at[idx_v[b]], rows_v[b], s_g[b]).wait()
            @pl.when(g + 1 < NCHUNK)
            def _():
                pltpu.make_async_copy(
                    idx_hbm.at[pl.ds((b0 + CB) * L, C)], idx_v[nb],
                    s_i[nb]).wait()
                pltpu.async_copy(w_hbm.at[idx_v[nb]], rows_v[nb], s_g[nb])

                @pl.when(g + 2 < NCHUNK)
                def _():
                    pltpu.async_copy(
                        idx_hbm.at[pl.ds((b0 + 2 * CB) * L, C)], idx_v[b],
                        s_i[b])

            transpose_half(b, 0, b0)
            transpose_half(b, 1, b0)
        return carry

    lax.fori_loop(0, NCHUNK // 2, body, 0)


def kernel(doc, W):
    idx2 = 2 * doc.reshape(BT)
    w2 = jnp.pad(W, ((0, 0), (0, D))).reshape(V2, D)
    out_t = _embed(idx2, w2)
    return jnp.transpose(out_t, (2, 0, 1))


# speedup vs baseline: 1.0007x; 1.0007x over previous
import functools

import jax
import jax.numpy as jnp
from jax import lax
from jax.experimental import pallas as pl
from jax.experimental.pallas import tpu as pltpu
from jax.experimental.pallas import tpu_sc as plsc

B = 16384
L = 50
D = 64
BT = B * L
V2 = 2000000

_info = plsc.get_sparse_core_info()
NC = _info.num_cores
NS = _info.num_subcores
NW = NC * NS
BPW_B = B // NW

CB = 16
C = CB * L
NCHUNK = BPW_B // CB
LH = L // 2

_mesh = plsc.VectorSubcoreMesh(core_axis_name="c", subcore_axis_name="s")


@functools.partial(
    pl.kernel,
    mesh=_mesh,
    out_type=jax.ShapeDtypeStruct((L * D, B), jnp.float32),
    scratch_types=[
        pltpu.VMEM((C,), jnp.int32),
        pltpu.VMEM((C,), jnp.int32),
        pltpu.VMEM((C, D), jnp.float32),
        pltpu.VMEM((C, D), jnp.float32),
        pltpu.VMEM((LH * D, CB), jnp.float32),
        pltpu.SemaphoreType.DMA,
        pltpu.SemaphoreType.DMA,
        pltpu.SemaphoreType.DMA,
        pltpu.SemaphoreType.DMA,
        pltpu.SemaphoreType.DMA,
    ],
    compiler_params=pltpu.CompilerParams(
        use_tc_tiling_on_sc=False, needs_layout_passes=False),
)
def _embed(idx_hbm, w_hbm, out_hbm,
           idx0, idx1, rows0, rows1, t_v, si0, si1, sg0, sg1, so):
    wid = lax.axis_index("s") * NC + lax.axis_index("c")
    base_b = wid * BPW_B
    idx_v = (idx0, idx1)
    rows_v = (rows0, rows1)
    s_i = (si0, si1)
    s_g = (sg0, sg1)

    lane = lax.iota(jnp.int32, 16)
    d_idx = [lane + 16 * k for k in range(4)]

    pltpu.sync_copy(idx_hbm.at[pl.ds(base_b * L, C)], idx0)
    pltpu.async_copy(w_hbm.at[idx0], rows0, sg0)
    pltpu.async_copy(idx_hbm.at[pl.ds(base_b * L + C, C)], idx1, si1)

    def transpose_half(buf, half, b0):
        def per_l(ll, carry):
            row_vecs = [d_idx[k] + ll * D for k in range(4)]
            l_glob = half * LH + ll
            for b_l in range(CB):
                r = b_l * L + l_glob
                b_idx = jnp.full((16,), b_l, jnp.int32)
                for k in range(4):
                    v = rows_v[buf][r, pl.ds(16 * k, 16)]
                    plsc.store_scatter(t_v, [row_vecs[k], b_idx], v)
            return carry

        lax.fori_loop(0, LH, per_l, 0)
        cp = pltpu.make_async_copy(
            t_v,
            out_hbm.at[pl.ds(half * LH * D, LH * D), pl.ds(b0, CB)],
            so)
        cp.start()
        cp.wait()

    def body(h, carry):
        for b in range(2):
            g = 2 * h + b
            b0 = base_b + g * CB
            nb = 1 - b
            pltpu.make_async_copy(w_hbm.at[idx_v[b]], rows_v[b], s_g[b]).wait()
            @pl.when(g + 1 < NCHUNK)
            def _():
                pltpu.make_async_copy(
                    idx_hbm.at[pl.ds((b0 + CB) * L, C)], idx_v[nb],
                    s_i[nb]).wait()
                pltpu.async_copy(w_hbm.at[idx_v[nb]], rows_v[nb], s_g[nb])

                @pl.when(g + 2 < NCHUNK)
                def _():
                    pltpu.async_copy(
                        idx_hbm.at[pl.ds((b0 + 2 * CB) * L, C)], idx_v[b],
                        s_i[b])

            transpose_half(b, 0, b0)
            transpose_half(b, 1, b0)
        return carry

    lax.fori_loop(0, NCHUNK // 2, body, 0)


def kernel(doc, W):
    idx2 = 2 * doc.reshape(BT)
    w2 = jnp.pad(W, ((0, 0), (0, D))).reshape(V2, D)
    out_t = _embed(idx2, w2)
    return jnp.transpose(out_t.reshape(L, D, B), (2, 0, 1))


# speedup vs baseline: 1.0942x; 1.0934x over previous
import functools

import jax
import jax.numpy as jnp
from jax import lax
from jax.experimental import pallas as pl
from jax.experimental.pallas import tpu as pltpu
from jax.experimental.pallas import tpu_sc as plsc

B = 16384
L = 50
D = 64
BT = B * L

_info = plsc.get_sparse_core_info()
NC = _info.num_cores
NS = _info.num_subcores
NW = NC * NS
BPW = BT // NW

C = 800
NCHUNK = BPW // C

_mesh = plsc.VectorSubcoreMesh(core_axis_name="c", subcore_axis_name="s")


@functools.partial(
    pl.kernel,
    mesh=_mesh,
    out_type=jax.ShapeDtypeStruct((BT, D), jnp.float32),
    scratch_types=[
        pltpu.VMEM((C,), jnp.int32),
        pltpu.VMEM((C,), jnp.int32),
        pltpu.VMEM((C, D), jnp.float32),
        pltpu.VMEM((C, D), jnp.float32),
        pltpu.SemaphoreType.DMA,
        pltpu.SemaphoreType.DMA,
        pltpu.SemaphoreType.DMA,
        pltpu.SemaphoreType.DMA,
        pltpu.SemaphoreType.DMA,
        pltpu.SemaphoreType.DMA,
    ],
    compiler_params=pltpu.CompilerParams(use_tc_tiling_on_sc=False),
)
def _embed(idx_hbm, w_hbm, out_hbm,
           idx0, idx1, rows0, rows1, si0, si1, sg0, sg1, so0, so1):
    wid = lax.axis_index("s") * NC + lax.axis_index("c")
    base = wid * BPW
    idx_v = (idx0, idx1)
    rows_v = (rows0, rows1)
    s_i = (si0, si1)
    s_g = (sg0, sg1)
    s_o = (so0, so1)

    pltpu.async_copy(idx_hbm.at[pl.ds(base, C)], idx0, si0)

    def body(h, carry):
        for b in range(2):
            g = 2 * h + b
            off = base + g * C
            nb = 1 - b
            pltpu.make_async_copy(
                idx_hbm.at[pl.ds(off, C)], idx_v[b], s_i[b]).wait()
            @pl.when(g >= 2)
            def _():
                pltpu.make_async_copy(
                    rows_v[b], out_hbm.at[pl.ds(off, C)], s_o[b]).wait()
            pltpu.async_copy(w_hbm.at[idx_v[b]], rows_v[b], s_g[b])
            @pl.when(g + 1 < NCHUNK)
            def _():
                pltpu.async_copy(
                    idx_hbm.at[pl.ds(off + C, C)], idx_v[nb], s_i[nb])
            pltpu.make_async_copy(w_hbm.at[idx_v[b]], rows_v[b], s_g[b]).wait()
            pltpu.async_copy(rows_v[b], out_hbm.at[pl.ds(off, C)], s_o[b])
        return carry

    lax.fori_loop(0, NCHUNK // 2, body, 0)

    pltpu.make_async_copy(rows0, out_hbm.at[pl.ds(base, C)], so0).wait()
    pltpu.make_async_copy(rows1, out_hbm.at[pl.ds(base, C)], so1).wait()


def kernel(doc, W):
    idx = doc.reshape(BT)
    out = _embed(idx, W)
    return out.reshape(B, L, D)
